# Initial kernel scaffold; baseline (speedup 1.0000x reference)
import jax
import jax.numpy as jnp
from jax.experimental import pallas as pl


def kernel(d_local, confidence, x, Wc, bc, Wi, bi, Wd, bd, Wo, bo):
    raise NotImplementedError("write your pallas kernel here")


# speedup vs baseline: 2.2559x; 2.2559x over previous
import functools

import jax
import jax.numpy as jnp
import numpy as np
from jax.experimental import pallas as pl
from jax.experimental.pallas import tpu as pltpu

WS = 8
N_THETA = 16
HIDDEN = 32
TOPK = 2
N_EXPERTS = 4

_INTERPRET = False
_HI = jax.lax.Precision.HIGHEST

_F = np.float32
_ERF_POLY = [_F("7.85386146e-05"), _F("-0.000801019371"), _F("0.00518832775"),
             _F("-0.0268538129"), _F("0.112835854"), _F("-0.37612626"),
             _F("1.12837911")]
_ERFC_P1 = [_F("0.0232682"), _F("-0.138703942"), _F("0.368742466"),
            _F("-0.582473278"), _F("0.621000469"), _F("-0.494451523"),
            _F("0.340488"), _F("-0.274112701"), _F("0.563825965")]
_ERFC_P2 = [_F("-10.477664"), _F("12.9772"), _F("-7.49551868"),
            _F("2.92101908"), _F("-1.01526523"), _F("0.42184633"),
            _F("-0.282076746"), _F("0.564189494")]


def _erfc(x):
    one = _F(1.0)
    ax = jnp.abs(x)
    z = x * x
    p = z * _ERF_POLY[0] + _ERF_POLY[1]
    for c in _ERF_POLY[2:]:
        p = p * z + c
    small = one - x * p
    nz = -z
    e = jnp.exp(nz)
    q = one / ax
    r = e * q
    w = one / z
    pa = w * _ERFC_P1[0] + _ERFC_P1[1]
    for c in _ERFC_P1[2:]:
        pa = pa * w + c
    pb = w * _ERFC_P2[0] + _ERFC_P2[1]
    for c in _ERFC_P2[2:]:
        pb = pb * w + c
    sel = jnp.where(ax < _F(2.0), pa, pb)
    big = r * sel
    big = jnp.where(nz < _F(-88.7228394), _F(0.0), big)
    big = jnp.where(x < _F(0.0), _F(2.0) - big, big)
    return jnp.where(ax < one, small, big)


def _pool_kernel(x_ref, p_ref, o_ref, *, th):
    xt = x_ref[0]
    c, rows, w = xt.shape
    xr = xt.reshape(c * th, WS, w)
    rowsum = xr[:, 0, :]
    for k in range(1, WS):
        rowsum = rowsum + xr[:, k, :]
    pooled = jnp.dot(rowsum, p_ref[...],
                     preferred_element_type=jnp.float32, precision=_HI)
    o_ref[0] = pooled.reshape(c, th, pooled.shape[1])


def _router_kernel(dl_ref, cf_ref, pool_ref, wc_ref, bc_ref, wi_ref, bi_ref,
                   wd_ref, bd_ref, wo_ref, bo_ref, alpha_ref):
    n = dl_ref.shape[2]
    hw = int(round(float(n) ** 0.5))

    ctx = jnp.dot(wc_ref[...], pool_ref[0],
                  preferred_element_type=jnp.float32) + bc_ref[...]
    ri = jnp.concatenate([dl_ref[0], cf_ref[0], ctx], axis=0)
    h = jnp.dot(wi_ref[...], ri,
                preferred_element_type=jnp.float32) + bi_ref[...]
    g = (h * _F(0.5)) * _erfc((-h) * _F(0.707106769))
    gb = g.astype(jnp.bfloat16).astype(jnp.float32)

    zpad = jnp.zeros((HIDDEN, 2 * hw), jnp.float32)
    gp = jnp.concatenate([zpad, gb, zpad], axis=1)
    wpos = jax.lax.broadcasted_iota(jnp.int32, (HIDDEN, n), 1) % hw
    not_left = wpos > 0
    not_right = wpos < hw - 1
    acc = None
    for di in range(3):
        for dj in range(3):
            dw = dj - 1
            off = 2 * hw + (di - 1) * hw + dw
            t = jax.lax.slice(gp, (0, off), (HIDDEN, off + n))
            if dw == -1:
                t = jnp.where(not_left, t, _F(0.0))
            elif dw == 1:
                t = jnp.where(not_right, t, _F(0.0))
            t = wd_ref[...][:, 3 * di + dj:3 * di + dj + 1] * t
            acc = t if acc is None else acc + t
    hd2 = acc + bd_ref[...]

    logits = jnp.dot(wo_ref[...], hd2,
                     preferred_element_type=jnp.float32) + bo_ref[...]

    ls = [logits[i] for i in range(N_EXPERTS)]
    m = ls[0]
    for i in range(1, N_EXPERTS):
        m = jnp.maximum(m, ls[i])
    es, s = [], None
    for i in range(N_EXPERTS):
        cnt = None
        for j in range(N_EXPERTS):
            if j == i:
                continue
            beats = (ls[j] >= ls[i]) if j < i else (ls[j] > ls[i])
            b32 = beats.astype(jnp.int32)
            cnt = b32 if cnt is None else cnt + b32
        e_i = jnp.where(cnt < TOPK, jnp.exp(ls[i] - m), _F(0.0))
        es.append(e_i)
        s = e_i if s is None else s + e_i
    alpha_ref[0] = jnp.stack([e / s for e in es], axis=0)


def _upsample_kernel(a_ref, cf_ref, e_ref, aup_ref, cup_ref):
    hw = a_ref.shape[2]
    em = e_ref[...]
    for e in range(N_EXPERTS):
        au = jnp.dot(a_ref[0, e], em,
                     preferred_element_type=jnp.float32, precision=_HI)
        au = jnp.broadcast_to(au[:, None, :], (hw, WS, WS * hw))
        aup_ref[0, e] = au.reshape(WS * hw, WS * hw)
    cu = jnp.dot(cf_ref[0, 0], em,
                 preferred_element_type=jnp.float32, precision=_HI)
    cu = jnp.broadcast_to(cu[:, None, :], (hw, WS, WS * hw))
    cup_ref[0, 0] = cu.reshape(WS * hw, WS * hw)


def kernel(d_local, confidence, x, Wc, bc, Wi, bi, Wd, bd, Wo, bo):
    B, C, H, W = x.shape
    Hc, Wc_ = H // WS, W // WS
    n = Hc * Wc_

    pool_mat = jnp.asarray(
        np.repeat(np.eye(Wc_, dtype=np.float32), WS, axis=0) / (WS * WS))
    up_mat = jnp.asarray(np.repeat(np.eye(Wc_, dtype=np.float32), WS, axis=1))

    th = 8
    pooled = pl.pallas_call(
        functools.partial(_pool_kernel, th=th),
        grid=(B, Hc // th),
        in_specs=[
            pl.BlockSpec((1, C, th * WS, W), lambda b, t: (b, 0, t, 0)),
            pl.BlockSpec((W, Wc_), lambda b, t: (0, 0)),
        ],
        out_specs=pl.BlockSpec((1, C, th, Wc_), lambda b, t: (b, 0, t, 0)),
        out_shape=jax.ShapeDtypeStruct((B, C, Hc, Wc_), jnp.float32),
        interpret=_INTERPRET,
    )(x, pool_mat)

    wd_bf = Wd.reshape(HIDDEN, 9).astype(jnp.bfloat16).astype(jnp.float32)

    alpha = pl.pallas_call(
        _router_kernel,
        grid=(B,),
        in_specs=[
            pl.BlockSpec((1, N_THETA, n), lambda b: (b, 0, 0)),
            pl.BlockSpec((1, 1, n), lambda b: (b, 0, 0)),
            pl.BlockSpec((1, C, n), lambda b: (b, 0, 0)),
            pl.BlockSpec((N_THETA, C), lambda b: (0, 0)),
            pl.BlockSpec((N_THETA, 1), lambda b: (0, 0)),
            pl.BlockSpec((HIDDEN, N_THETA * 2 + 1), lambda b: (0, 0)),
            pl.BlockSpec((HIDDEN, 1), lambda b: (0, 0)),
            pl.BlockSpec((HIDDEN, 9), lambda b: (0, 0)),
            pl.BlockSpec((HIDDEN, 1), lambda b: (0, 0)),
            pl.BlockSpec((N_EXPERTS, HIDDEN), lambda b: (0, 0)),
            pl.BlockSpec((N_EXPERTS, 1), lambda b: (0, 0)),
        ],
        out_specs=pl.BlockSpec((1, N_EXPERTS, n), lambda b: (b, 0, 0)),
        out_shape=jax.ShapeDtypeStruct((B, N_EXPERTS, n), jnp.float32),
        interpret=_INTERPRET,
    )(d_local.reshape(B, N_THETA, n), confidence.reshape(B, 1, n),
      pooled.reshape(B, C, n), Wc, bc[:, None], Wi, bi[:, None],
      wd_bf, bd[:, None], Wo, bo[:, None])

    alpha4 = alpha.reshape(B, N_EXPERTS, Hc, Wc_)
    alpha_up, conf_up = pl.pallas_call(
        _upsample_kernel,
        grid=(B,),
        in_specs=[
            pl.BlockSpec((1, N_EXPERTS, Hc, Wc_), lambda b: (b, 0, 0, 0)),
            pl.BlockSpec((1, 1, Hc, Wc_), lambda b: (b, 0, 0, 0)),
            pl.BlockSpec((Wc_, W), lambda b: (0, 0)),
        ],
        out_specs=[
            pl.BlockSpec((1, N_EXPERTS, H, W), lambda b: (b, 0, 0, 0)),
            pl.BlockSpec((1, 1, H, W), lambda b: (b, 0, 0, 0)),
        ],
        out_shape=[
            jax.ShapeDtypeStruct((B, N_EXPERTS, H, W), jnp.float32),
            jax.ShapeDtypeStruct((B, 1, H, W), jnp.float32),
        ],
        interpret=_INTERPRET,
    )(alpha4, confidence, up_mat)

    return alpha4, alpha_up, conf_up


# speedup vs baseline: 3.7010x; 1.6406x over previous
import functools

import jax
import jax.numpy as jnp
import numpy as np
from jax.experimental import pallas as pl
from jax.experimental.pallas import tpu as pltpu

WS = 8
N_THETA = 16
HIDDEN = 32
TOPK = 2
N_EXPERTS = 4

_INTERPRET = False
_HI = jax.lax.Precision.HIGHEST

_F = np.float32
_ERF_POLY = [_F("7.85386146e-05"), _F("-0.000801019371"), _F("0.00518832775"),
             _F("-0.0268538129"), _F("0.112835854"), _F("-0.37612626"),
             _F("1.12837911")]
_ERFC_P1 = [_F("0.0232682"), _F("-0.138703942"), _F("0.368742466"),
            _F("-0.582473278"), _F("0.621000469"), _F("-0.494451523"),
            _F("0.340488"), _F("-0.274112701"), _F("0.563825965")]
_ERFC_P2 = [_F("-10.477664"), _F("12.9772"), _F("-7.49551868"),
            _F("2.92101908"), _F("-1.01526523"), _F("0.42184633"),
            _F("-0.282076746"), _F("0.564189494")]


def _erfc(x):
    one = _F(1.0)
    ax = jnp.abs(x)
    z = x * x
    p = z * _ERF_POLY[0] + _ERF_POLY[1]
    for c in _ERF_POLY[2:]:
        p = p * z + c
    small = one - x * p
    nz = -z
    e = jnp.exp(nz)
    q = one / ax
    r = e * q
    w = one / z
    pa = w * _ERFC_P1[0] + _ERFC_P1[1]
    for c in _ERFC_P1[2:]:
        pa = pa * w + c
    pb = w * _ERFC_P2[0] + _ERFC_P2[1]
    for c in _ERFC_P2[2:]:
        pb = pb * w + c
    sel = jnp.where(ax < _F(2.0), pa, pb)
    big = r * sel
    big = jnp.where(nz < _F(-88.7228394), _F(0.0), big)
    big = jnp.where(x < _F(0.0), _F(2.0) - big, big)
    return jnp.where(ax < one, small, big)


def _pool_kernel(x_ref, o_ref, *, th):
    xt = x_ref[0]
    c, rows, w = xt.shape
    xr = xt.reshape(c * th, WS, w)
    rowsum = xr.sum(axis=1)
    rt = rowsum.T.reshape(w // WS, WS, c * th).sum(axis=1)
    pooled = rt.T * _F(1.0 / (WS * WS))
    o_ref[0] = pooled.reshape(c, th, pooled.shape[1])


def _router_kernel(dl_ref, cf_ref, pool_ref, wc_ref, bc_ref, wi_ref, bi_ref,
                   wd_ref, bd_ref, wo_ref, bo_ref, alpha_ref):
    n = dl_ref.shape[2]
    hw = int(round(float(n) ** 0.5))

    ctx = jnp.dot(wc_ref[...], pool_ref[0],
                  preferred_element_type=jnp.float32) + bc_ref[...]
    ri = jnp.concatenate([dl_ref[0], cf_ref[0], ctx], axis=0)
    h = jnp.dot(wi_ref[...], ri,
                preferred_element_type=jnp.float32) + bi_ref[...]
    g = (h * _F(0.5)) * _erfc((-h) * _F(0.707106769))
    gb = g.astype(jnp.bfloat16).astype(jnp.float32)

    zpad = jnp.zeros((HIDDEN, 2 * hw), jnp.float32)
    gp = jnp.concatenate([zpad, gb, zpad], axis=1)
    wpos = jax.lax.broadcasted_iota(jnp.int32, (HIDDEN, n), 1) % hw
    not_left = wpos > 0
    not_right = wpos < hw - 1
    acc = None
    for di in range(3):
        for dj in range(3):
            dw = dj - 1
            off = 2 * hw + (di - 1) * hw + dw
            t = jax.lax.slice(gp, (0, off), (HIDDEN, off + n))
            if dw == -1:
                t = jnp.where(not_left, t, _F(0.0))
            elif dw == 1:
                t = jnp.where(not_right, t, _F(0.0))
            t = wd_ref[...][:, 3 * di + dj:3 * di + dj + 1] * t
            acc = t if acc is None else acc + t
    hd2 = acc + bd_ref[...]

    logits = jnp.dot(wo_ref[...], hd2,
                     preferred_element_type=jnp.float32) + bo_ref[...]

    ls = [logits[i] for i in range(N_EXPERTS)]
    m = ls[0]
    for i in range(1, N_EXPERTS):
        m = jnp.maximum(m, ls[i])
    es, s = [], None
    for i in range(N_EXPERTS):
        cnt = None
        for j in range(N_EXPERTS):
            if j == i:
                continue
            beats = (ls[j] >= ls[i]) if j < i else (ls[j] > ls[i])
            b32 = beats.astype(jnp.int32)
            cnt = b32 if cnt is None else cnt + b32
        e_i = jnp.where(cnt < TOPK, jnp.exp(ls[i] - m), _F(0.0))
        es.append(e_i)
        s = e_i if s is None else s + e_i
    alpha_ref[0] = jnp.stack([e / s for e in es], axis=0)


def _upsample_kernel(a_ref, cf_ref, e_ref, aup_ref, cup_ref):
    hw = a_ref.shape[2]
    em = e_ref[...]
    for e in range(N_EXPERTS):
        au = jnp.dot(a_ref[0, e], em,
                     preferred_element_type=jnp.float32, precision=_HI)
        au = jnp.broadcast_to(au[:, None, :], (hw, WS, WS * hw))
        aup_ref[0, e] = au.reshape(WS * hw, WS * hw)
    cu = jnp.dot(cf_ref[0, 0], em,
                 preferred_element_type=jnp.float32, precision=_HI)
    cu = jnp.broadcast_to(cu[:, None, :], (hw, WS, WS * hw))
    cup_ref[0, 0] = cu.reshape(WS * hw, WS * hw)


def kernel(d_local, confidence, x, Wc, bc, Wi, bi, Wd, bd, Wo, bo):
    B, C, H, W = x.shape
    Hc, Wc_ = H // WS, W // WS
    n = Hc * Wc_

    pool_mat = jnp.asarray(
        np.repeat(np.eye(Wc_, dtype=np.float32), WS, axis=0) / (WS * WS))
    up_mat = jnp.asarray(np.repeat(np.eye(Wc_, dtype=np.float32), WS, axis=1))

    th = 8
    pooled = pl.pallas_call(
        functools.partial(_pool_kernel, th=th),
        grid=(B, Hc // th),
        in_specs=[
            pl.BlockSpec((1, C, th * WS, W), lambda b, t: (b, 0, t, 0)),
        ],
        out_specs=pl.BlockSpec((1, C, th, Wc_), lambda b, t: (b, 0, t, 0)),
        out_shape=jax.ShapeDtypeStruct((B, C, Hc, Wc_), jnp.float32),
        interpret=_INTERPRET,
    )(x)

    wd_bf = Wd.reshape(HIDDEN, 9).astype(jnp.bfloat16).astype(jnp.float32)

    alpha = pl.pallas_call(
        _router_kernel,
        grid=(B,),
        in_specs=[
            pl.BlockSpec((1, N_THETA, n), lambda b: (b, 0, 0)),
            pl.BlockSpec((1, 1, n), lambda b: (b, 0, 0)),
            pl.BlockSpec((1, C, n), lambda b: (b, 0, 0)),
            pl.BlockSpec((N_THETA, C), lambda b: (0, 0)),
            pl.BlockSpec((N_THETA, 1), lambda b: (0, 0)),
            pl.BlockSpec((HIDDEN, N_THETA * 2 + 1), lambda b: (0, 0)),
            pl.BlockSpec((HIDDEN, 1), lambda b: (0, 0)),
            pl.BlockSpec((HIDDEN, 9), lambda b: (0, 0)),
            pl.BlockSpec((HIDDEN, 1), lambda b: (0, 0)),
            pl.BlockSpec((N_EXPERTS, HIDDEN), lambda b: (0, 0)),
            pl.BlockSpec((N_EXPERTS, 1), lambda b: (0, 0)),
        ],
        out_specs=pl.BlockSpec((1, N_EXPERTS, n), lambda b: (b, 0, 0)),
        out_shape=jax.ShapeDtypeStruct((B, N_EXPERTS, n), jnp.float32),
        interpret=_INTERPRET,
    )(d_local.reshape(B, N_THETA, n), confidence.reshape(B, 1, n),
      pooled.reshape(B, C, n), Wc, bc[:, None], Wi, bi[:, None],
      wd_bf, bd[:, None], Wo, bo[:, None])

    alpha4 = alpha.reshape(B, N_EXPERTS, Hc, Wc_)
    alpha_up, conf_up = pl.pallas_call(
        _upsample_kernel,
        grid=(B,),
        in_specs=[
            pl.BlockSpec((1, N_EXPERTS, Hc, Wc_), lambda b: (b, 0, 0, 0)),
            pl.BlockSpec((1, 1, Hc, Wc_), lambda b: (b, 0, 0, 0)),
            pl.BlockSpec((Wc_, W), lambda b: (0, 0)),
        ],
        out_specs=[
            pl.BlockSpec((1, N_EXPERTS, H, W), lambda b: (b, 0, 0, 0)),
            pl.BlockSpec((1, 1, H, W), lambda b: (b, 0, 0, 0)),
        ],
        out_shape=[
            jax.ShapeDtypeStruct((B, N_EXPERTS, H, W), jnp.float32),
            jax.ShapeDtypeStruct((B, 1, H, W), jnp.float32),
        ],
        interpret=_INTERPRET,
    )(alpha4, confidence, up_mat)

    return alpha4, alpha_up, conf_up
